# Initial kernel scaffold; baseline (speedup 1.0000x reference)
import jax
import jax.numpy as jnp
from jax.experimental import pallas as pl


def kernel(x, edge_index, edge_weight, Wc0, bc0, Wc1, bc1, Wc2, bc2, Wc3, bc3, Wc4, bc4, Wn0, bn0, Wn1, bn1, We0, be0, We1, be1, We2, be2, We3, be3):
    raise NotImplementedError("write your pallas kernel here")


# speedup vs baseline: 5.0641x; 5.0641x over previous
import functools

import jax
import jax.numpy as jnp
from jax import lax
from jax.experimental import pallas as pl
from jax.experimental.pallas import tpu as pltpu
from jax.experimental.pallas import tpu_sc as plsc

N = 10000
E = 320000
NP = 10240
NC, NS, L = 2, 16, 16
NW = NC * NS
EPW = E // NW
CH = 80
NCHUNK = EPW // CH
RPT = NP // NS

_mesh = lambda: plsc.VectorSubcoreMesh(core_axis_name="c", subcore_axis_name="s")


def _deg_kernel_fn():
    @functools.partial(
        pl.kernel,
        out_type=jax.ShapeDtypeStruct((NC, NP), jnp.float32),
        mesh=_mesh(),
        scratch_types=[
            pltpu.VMEM((CH,), jnp.int32),
            pltpu.VMEM((CH,), jnp.float32),
            pltpu.VMEM((RPT,), jnp.float32),
            pltpu.VMEM_SHARED((NP,), jnp.float32),
        ],
    )
    def k(didx_hbm, w_hbm, out_hbm, didx_v, w_v, z_v, acc_sh):
        c = lax.axis_index("c")
        s = lax.axis_index("s")
        w = s * NC + c

        def zb(i, _):
            z_v[pl.ds(i * L, L)] = jnp.zeros((L,), jnp.float32)
            return 0
        lax.fori_loop(0, RPT // L, zb, 0)
        pltpu.sync_copy(z_v, acc_sh.at[pl.ds(s * RPT, RPT)])
        plsc.subcore_barrier()

        base = w * EPW

        def body(i, _):
            off = base + i * CH
            pltpu.sync_copy(didx_hbm.at[pl.ds(off, CH)], didx_v)
            pltpu.sync_copy(w_hbm.at[pl.ds(off, CH)], w_v)
            pltpu.sync_copy(w_v, acc_sh.at[didx_v], add=True)
            return 0
        lax.fori_loop(0, NCHUNK, body, 0)

        plsc.subcore_barrier()
        pltpu.sync_copy(acc_sh.at[pl.ds(s * RPT, RPT)],
                        out_hbm.at[c, pl.ds(s * RPT, RPT)])

    return k


def _norm_kernel_fn():
    @functools.partial(
        pl.kernel,
        out_type=jax.ShapeDtypeStruct((E,), jnp.float32),
        mesh=_mesh(),
        scratch_types=[
            pltpu.VMEM((CH,), jnp.int32),
            pltpu.VMEM((CH,), jnp.int32),
            pltpu.VMEM((CH,), jnp.float32),
            pltpu.VMEM((CH,), jnp.float32),
            pltpu.VMEM((CH,), jnp.float32),
            pltpu.VMEM((CH,), jnp.float32),
            pltpu.SemaphoreType.DMA,
        ],
    )
    def k(dis_hbm, sidx_hbm, didx_hbm, w_hbm, out_hbm,
          sidx_v, didx_v, w_v, dss_v, dsd_v, o_v, sem):
        c = lax.axis_index("c")
        s = lax.axis_index("s")
        wkr = s * NC + c
        base = wkr * EPW

        def body(i, _):
            off = base + i * CH
            pltpu.sync_copy(sidx_hbm.at[pl.ds(off, CH)], sidx_v)
            pltpu.sync_copy(didx_hbm.at[pl.ds(off, CH)], didx_v)
            pltpu.sync_copy(w_hbm.at[pl.ds(off, CH)], w_v)
            pltpu.async_copy(dis_hbm.at[sidx_v], dss_v, sem).wait()
            pltpu.async_copy(dis_hbm.at[didx_v], dsd_v, sem).wait()

            def inner(j, _):
                sl = pl.ds(j * L, L)
                o_v[sl] = dss_v[sl] * w_v[sl] * dsd_v[sl]
                return 0
            lax.fori_loop(0, CH // L, inner, 0)
            pltpu.sync_copy(o_v, out_hbm.at[pl.ds(off, CH)])
            return 0
        lax.fori_loop(0, NCHUNK, body, 0)

    return k


def _scatter_kernel_fn(O):
    @functools.partial(
        pl.kernel,
        out_type=jax.ShapeDtypeStruct((NC, NP, O), jnp.float32),
        mesh=_mesh(),
        scratch_types=[
            pltpu.VMEM((CH,), jnp.int32),
            pltpu.VMEM((CH,), jnp.int32),
            pltpu.VMEM((CH + L,), jnp.float32),
            pltpu.VMEM((CH, O), jnp.float32),
            pltpu.VMEM_SHARED((NP, O), jnp.float32),
            pltpu.SemaphoreType.DMA,
        ],
        compiler_params=pltpu.CompilerParams(use_tc_tiling_on_sc=False),
    )
    def k(hw_hbm, sidx_hbm, didx_hbm, nrm_hbm, out_hbm,
          sidx_v, didx_v, nrm_v, rows_v, acc_sh, sem):
        c = lax.axis_index("c")
        s = lax.axis_index("s")
        wkr = s * NC + c
        r0 = s * RPT

        def zrow(i, _):
            def zcol(f, _):
                rows_v[i, pl.ds(f * L, L)] = jnp.zeros((L,), jnp.float32)
                return 0
            lax.fori_loop(0, O // L, zcol, 0)
            return 0
        lax.fori_loop(0, CH, zrow, 0)
        for t in range(RPT // CH):
            pltpu.sync_copy(rows_v, acc_sh.at[pl.ds(r0 + t * CH, CH)])
        plsc.subcore_barrier()

        base = wkr * EPW

        def body(i, _):
            off = base + i * CH
            pltpu.sync_copy(sidx_hbm.at[pl.ds(off, CH)], sidx_v)
            pltpu.sync_copy(didx_hbm.at[pl.ds(off, CH)], didx_v)
            pltpu.sync_copy(nrm_hbm.at[pl.ds(off, CH)], nrm_v.at[pl.ds(0, CH)])
            pltpu.async_copy(hw_hbm.at[sidx_v], rows_v, sem).wait()

            def ebody(j, _):
                nv16 = nrm_v[pl.ds(j, L)]
                nv = jnp.full((L,), nv16[0], jnp.float32)
                for f in range(O // L):
                    rows_v[j, pl.ds(f * L, L)] = rows_v[j, pl.ds(f * L, L)] * nv
                return 0
            lax.fori_loop(0, CH, ebody, 0)

            pltpu.sync_copy(rows_v, acc_sh.at[didx_v], add=True)
            return 0
        lax.fori_loop(0, NCHUNK, body, 0)

        plsc.subcore_barrier()
        for t in range(RPT // CH):
            pltpu.sync_copy(acc_sh.at[pl.ds(r0 + t * CH, CH)],
                            out_hbm.at[c, pl.ds(r0 + t * CH, CH)])

    return k


def _edge_gather_kernel_fn(O):
    @functools.partial(
        pl.kernel,
        out_type=(jax.ShapeDtypeStruct((E, O), jnp.float32),
                  jax.ShapeDtypeStruct((E, O), jnp.float32)),
        mesh=_mesh(),
        scratch_types=[
            pltpu.VMEM((CH,), jnp.int32),
            pltpu.VMEM((CH, O), jnp.float32),
            pltpu.SemaphoreType.DMA,
        ],
        compiler_params=pltpu.CompilerParams(use_tc_tiling_on_sc=False),
    )
    def k(h_hbm, sidx_hbm, didx_hbm, xs_hbm, xd_hbm, idx_v, rows_v, sem):
        c = lax.axis_index("c")
        s = lax.axis_index("s")
        wkr = s * NC + c
        base = wkr * EPW

        def body(i, _):
            off = base + i * CH
            pltpu.sync_copy(sidx_hbm.at[pl.ds(off, CH)], idx_v)
            pltpu.async_copy(h_hbm.at[idx_v], rows_v, sem).wait()
            pltpu.sync_copy(rows_v, xs_hbm.at[pl.ds(off, CH)])
            pltpu.sync_copy(didx_hbm.at[pl.ds(off, CH)], idx_v)
            pltpu.async_copy(h_hbm.at[idx_v], rows_v, sem).wait()
            pltpu.sync_copy(rows_v, xd_hbm.at[pl.ds(off, CH)])
            return 0
        lax.fori_loop(0, NCHUNK, body, 0)

    return k


RB = 1024
RE = 2000


def _dis_pallas(deg0, deg1):
    def body(d0, d1, dis_o, sn_o):
        deg = d0[...] + d1[...] + 1.0
        dis = jnp.where(deg > 0, lax.rsqrt(deg), 0.0)
        dis_o[...] = dis
        sn_o[...] = dis * dis
    return pl.pallas_call(
        body,
        out_shape=(jax.ShapeDtypeStruct((NP // 128, 128), jnp.float32),
                   jax.ShapeDtypeStruct((NP // 128, 128), jnp.float32)),
    )(deg0.reshape(NP // 128, 128), deg1.reshape(NP // 128, 128))


def _mm0_pallas(x, W):
    C, O = W.shape[1], W.shape[0]

    def body(x_ref, w_ref, o_ref):
        o_ref[...] = lax.dot_general(
            x_ref[...], w_ref[...], (((1,), (1,)), ((), ())),
            preferred_element_type=jnp.float32)

    return pl.pallas_call(
        body,
        grid=(NP // RB,),
        in_specs=[pl.BlockSpec((RB, C), lambda i: (i, 0)),
                  pl.BlockSpec((O, C), lambda i: (0, 0))],
        out_specs=pl.BlockSpec((RB, O), lambda i: (i, 0)),
        out_shape=jax.ShapeDtypeStruct((NP, O), jnp.float32),
    )(x, W)


def _fused_layer_pallas(a0, a1, hw, sn, b, Wnext):
    O = hw.shape[1]
    On = Wnext.shape[0]

    def body(a0_ref, a1_ref, hw_ref, sn_ref, b_ref, w_ref, o_ref):
        h = a0_ref[...] + a1_ref[...] + hw_ref[...] * sn_ref[...] + b_ref[...]
        h = jnp.maximum(h, 0.0)
        o_ref[...] = lax.dot_general(
            h, w_ref[...], (((1,), (1,)), ((), ())),
            preferred_element_type=jnp.float32)

    return pl.pallas_call(
        body,
        grid=(NP // RB,),
        in_specs=[pl.BlockSpec((RB, O), lambda i: (i, 0)),
                  pl.BlockSpec((RB, O), lambda i: (i, 0)),
                  pl.BlockSpec((RB, O), lambda i: (i, 0)),
                  pl.BlockSpec((RB, 1), lambda i: (i, 0)),
                  pl.BlockSpec((1, O), lambda i: (0, 0)),
                  pl.BlockSpec((On, O), lambda i: (0, 0))],
        out_specs=pl.BlockSpec((RB, On), lambda i: (i, 0)),
        out_shape=jax.ShapeDtypeStruct((NP, On), jnp.float32),
    )(a0, a1, hw, sn, b, Wnext)


def _final_layer_pallas(a0, a1, hw, sn, b, Wn0, bn0, Wn1, bn1):
    O = hw.shape[1]

    def body(a0_ref, a1_ref, hw_ref, sn_ref, b_ref,
             wn0_ref, bn0_ref, wn1_ref, bn1_ref, h_o, n_o):
        h = a0_ref[...] + a1_ref[...] + hw_ref[...] * sn_ref[...] + b_ref[...]
        h = jnp.maximum(h, 0.0)
        h_o[...] = h
        n1 = lax.dot_general(h, wn0_ref[...], (((1,), (1,)), ((), ())),
                             preferred_element_type=jnp.float32)
        n1 = jnp.maximum(n1 + bn0_ref[...], 0.0)
        n2 = lax.dot_general(n1, wn1_ref[...], (((1,), (1,)), ((), ())),
                             preferred_element_type=jnp.float32)
        n_o[...] = jnp.maximum(n2 + bn1_ref[0], 0.0)

    return pl.pallas_call(
        body,
        grid=(NP // RB,),
        in_specs=[pl.BlockSpec((RB, O), lambda i: (i, 0)),
                  pl.BlockSpec((RB, O), lambda i: (i, 0)),
                  pl.BlockSpec((RB, O), lambda i: (i, 0)),
                  pl.BlockSpec((RB, 1), lambda i: (i, 0)),
                  pl.BlockSpec((1, O), lambda i: (0, 0)),
                  pl.BlockSpec((32, 32), lambda i: (0, 0)),
                  pl.BlockSpec((1, 32), lambda i: (0, 0)),
                  pl.BlockSpec((8, 32), lambda i: (0, 0)),
                  pl.BlockSpec(memory_space=pltpu.SMEM)],
        out_specs=(pl.BlockSpec((RB, O), lambda i: (i, 0)),
                   pl.BlockSpec((RB, 8), lambda i: (i, 0))),
        out_shape=(jax.ShapeDtypeStruct((NP, O), jnp.float32),
                   jax.ShapeDtypeStruct((NP, 8), jnp.float32)),
    )(a0, a1, hw, sn, b, Wn0, bn0, Wn1, bn1)


def _edge_mlp_pallas(xs, xd, A1, B1, be0, We1, be1, We2, be2, We3, be3):
    def body(xs_ref, xd_ref, a1_ref, b1_ref, be0_ref, w1_ref, b1b_ref,
             w2_ref, b2_ref, w3_ref, b3_ref, o_ref):
        e = lax.dot_general(xs_ref[...], a1_ref[...], (((1,), (1,)), ((), ())),
                            preferred_element_type=jnp.float32)
        e = e + lax.dot_general(xd_ref[...], b1_ref[...],
                                (((1,), (1,)), ((), ())),
                                preferred_element_type=jnp.float32)
        e = jnp.maximum(e + be0_ref[...], 0.0)
        e = lax.dot_general(e, w1_ref[...], (((1,), (1,)), ((), ())),
                            preferred_element_type=jnp.float32)
        e = jnp.maximum(e + b1b_ref[...], 0.0)
        e = lax.dot_general(e, w2_ref[...], (((1,), (1,)), ((), ())),
                            preferred_element_type=jnp.float32)
        e = jnp.maximum(e + b2_ref[...], 0.0)
        e = lax.dot_general(e, w3_ref[...], (((1,), (1,)), ((), ())),
                            preferred_element_type=jnp.float32)
        o_ref[...] = jnp.maximum(e + b3_ref[0], 0.0)

    return pl.pallas_call(
        body,
        grid=(E // RE,),
        in_specs=[pl.BlockSpec((RE, 32), lambda i: (i, 0)),
                  pl.BlockSpec((RE, 32), lambda i: (i, 0)),
                  pl.BlockSpec((64, 32), lambda i: (0, 0)),
                  pl.BlockSpec((64, 32), lambda i: (0, 0)),
                  pl.BlockSpec((1, 64), lambda i: (0, 0)),
                  pl.BlockSpec((32, 64), lambda i: (0, 0)),
                  pl.BlockSpec((1, 32), lambda i: (0, 0)),
                  pl.BlockSpec((32, 32), lambda i: (0, 0)),
                  pl.BlockSpec((1, 32), lambda i: (0, 0)),
                  pl.BlockSpec((8, 32), lambda i: (0, 0)),
                  pl.BlockSpec(memory_space=pltpu.SMEM)],
        out_specs=pl.BlockSpec((RE, 8), lambda i: (i, 0)),
        out_shape=jax.ShapeDtypeStruct((E, 8), jnp.float32),
    )(xs, xd, A1, B1, be0, We1, be1, We2, be2, We3, be3)


def kernel(x, edge_index, edge_weight,
           Wc0, bc0, Wc1, bc1, Wc2, bc2, Wc3, bc3, Wc4, bc4,
           Wn0, bn0, Wn1, bn1,
           We0, be0, We1, be1, We2, be2, We3, be3):
    src = edge_index[0]
    dst = edge_index[1]

    deg = _deg_kernel_fn()(dst, edge_weight)
    dis2d, sn2d = _dis_pallas(deg[0], deg[1])
    dis = dis2d.reshape(NP)
    sn = sn2d.reshape(NP, 1)
    nrm = _norm_kernel_fn()(dis, src, dst, edge_weight)

    xp = jnp.pad(x, ((0, NP - N), (0, 0)))

    convs = ((Wc0, bc0), (Wc1, bc1), (Wc2, bc2), (Wc3, bc3), (Wc4, bc4))
    hw = _mm0_pallas(xp, Wc0)
    for li in range(5):
        W, b = convs[li]
        O = W.shape[0]
        acc = _scatter_kernel_fn(O)(hw, src, dst, nrm)
        if li < 4:
            Wnext = convs[li + 1][0]
            hw = _fused_layer_pallas(acc[0], acc[1], hw, sn,
                                     b.reshape(1, O), Wnext)
        else:
            h5, n_full = _final_layer_pallas(
                acc[0], acc[1], hw, sn, b.reshape(1, O),
                Wn0, bn0.reshape(1, 32), jnp.pad(Wn1, ((0, 7), (0, 0))), bn1)

    xs, xd = _edge_gather_kernel_fn(32)(h5, src, dst)
    e = _edge_mlp_pallas(xs, xd,
                         We0[:, :32], We0[:, 32:], be0.reshape(1, 64),
                         We1, be1.reshape(1, 32),
                         We2, be2.reshape(1, 32),
                         jnp.pad(We3, ((0, 7), (0, 0))), be3)
    return (n_full[:N, :1], e[:, :1])


# speedup vs baseline: 5.6172x; 1.1092x over previous
import functools

import jax
import jax.numpy as jnp
from jax import lax
from jax.experimental import pallas as pl
from jax.experimental.pallas import tpu as pltpu
from jax.experimental.pallas import tpu_sc as plsc

N = 10000
E = 320000
NP = 10240
NC, NS, L = 2, 16, 16
NW = NC * NS
EPW = E // NW
CH = 80
NCHUNK = EPW // CH
RPT = NP // NS

_mesh = lambda: plsc.VectorSubcoreMesh(core_axis_name="c", subcore_axis_name="s")


def _deg_kernel_fn():
    @functools.partial(
        pl.kernel,
        out_type=jax.ShapeDtypeStruct((NC, NP), jnp.float32),
        mesh=_mesh(),
        scratch_types=[
            pltpu.VMEM((CH,), jnp.int32),
            pltpu.VMEM((CH,), jnp.float32),
            pltpu.VMEM((RPT,), jnp.float32),
            pltpu.VMEM_SHARED((NP,), jnp.float32),
        ],
    )
    def k(didx_hbm, w_hbm, out_hbm, didx_v, w_v, z_v, acc_sh):
        c = lax.axis_index("c")
        s = lax.axis_index("s")
        w = s * NC + c

        def zb(i, _):
            z_v[pl.ds(i * L, L)] = jnp.zeros((L,), jnp.float32)
            return 0
        lax.fori_loop(0, RPT // L, zb, 0)
        pltpu.sync_copy(z_v, acc_sh.at[pl.ds(s * RPT, RPT)])
        plsc.subcore_barrier()

        base = w * EPW

        def body(i, _):
            off = base + i * CH
            pltpu.sync_copy(didx_hbm.at[pl.ds(off, CH)], didx_v)
            pltpu.sync_copy(w_hbm.at[pl.ds(off, CH)], w_v)
            pltpu.sync_copy(w_v, acc_sh.at[didx_v], add=True)
            return 0
        lax.fori_loop(0, NCHUNK, body, 0)

        plsc.subcore_barrier()
        pltpu.sync_copy(acc_sh.at[pl.ds(s * RPT, RPT)],
                        out_hbm.at[c, pl.ds(s * RPT, RPT)])

    return k


def _scatter_kernel_fn(O):
    @functools.partial(
        pl.kernel,
        out_type=jax.ShapeDtypeStruct((NC, NP, O), jnp.float32),
        mesh=_mesh(),
        scratch_types=[
            pltpu.VMEM((CH,), jnp.int32),
            pltpu.VMEM((CH,), jnp.int32),
            pltpu.VMEM((CH + L,), jnp.float32),
            pltpu.VMEM((CH, O), jnp.float32),
            pltpu.VMEM_SHARED((NP, O), jnp.float32),
            pltpu.SemaphoreType.DMA,
        ],
        compiler_params=pltpu.CompilerParams(use_tc_tiling_on_sc=False),
    )
    def k(hw_hbm, sidx_hbm, didx_hbm, nrm_hbm, out_hbm,
          sidx_v, didx_v, nrm_v, rows_v, acc_sh, sem):
        c = lax.axis_index("c")
        s = lax.axis_index("s")
        wkr = s * NC + c
        r0 = s * RPT

        def zrow(i, _):
            def zcol(f, _):
                rows_v[i, pl.ds(f * L, L)] = jnp.zeros((L,), jnp.float32)
                return 0
            lax.fori_loop(0, O // L, zcol, 0)
            return 0
        lax.fori_loop(0, CH, zrow, 0)
        for t in range(RPT // CH):
            pltpu.sync_copy(rows_v, acc_sh.at[pl.ds(r0 + t * CH, CH)])
        plsc.subcore_barrier()

        base = wkr * EPW

        def body(i, _):
            off = base + i * CH
            pltpu.sync_copy(sidx_hbm.at[pl.ds(off, CH)], sidx_v)
            pltpu.sync_copy(didx_hbm.at[pl.ds(off, CH)], didx_v)
            pltpu.sync_copy(nrm_hbm.at[pl.ds(off, CH)], nrm_v.at[pl.ds(0, CH)])
            pltpu.async_copy(hw_hbm.at[sidx_v], rows_v, sem).wait()

            def ebody(j, _):
                nv16 = nrm_v[pl.ds(j, L)]
                nv = jnp.full((L,), nv16[0], jnp.float32)
                for f in range(O // L):
                    rows_v[j, pl.ds(f * L, L)] = rows_v[j, pl.ds(f * L, L)] * nv
                return 0
            lax.fori_loop(0, CH, ebody, 0)

            pltpu.sync_copy(rows_v, acc_sh.at[didx_v], add=True)
            return 0
        lax.fori_loop(0, NCHUNK, body, 0)

        plsc.subcore_barrier()
        for t in range(RPT // CH):
            pltpu.sync_copy(acc_sh.at[pl.ds(r0 + t * CH, CH)],
                            out_hbm.at[c, pl.ds(r0 + t * CH, CH)])

    return k


def _edge_gather_kernel_fn(O):
    @functools.partial(
        pl.kernel,
        out_type=(jax.ShapeDtypeStruct((E, O), jnp.float32),
                  jax.ShapeDtypeStruct((E, O), jnp.float32)),
        mesh=_mesh(),
        scratch_types=[
            pltpu.VMEM((CH,), jnp.int32),
            pltpu.VMEM((CH, O), jnp.float32),
            pltpu.SemaphoreType.DMA,
        ],
        compiler_params=pltpu.CompilerParams(use_tc_tiling_on_sc=False),
    )
    def k(h_hbm, sidx_hbm, didx_hbm, xs_hbm, xd_hbm, idx_v, rows_v, sem):
        c = lax.axis_index("c")
        s = lax.axis_index("s")
        wkr = s * NC + c
        base = wkr * EPW

        def body(i, _):
            off = base + i * CH
            pltpu.sync_copy(sidx_hbm.at[pl.ds(off, CH)], idx_v)
            pltpu.async_copy(h_hbm.at[idx_v], rows_v, sem).wait()
            pltpu.sync_copy(rows_v, xs_hbm.at[pl.ds(off, CH)])
            pltpu.sync_copy(didx_hbm.at[pl.ds(off, CH)], idx_v)
            pltpu.async_copy(h_hbm.at[idx_v], rows_v, sem).wait()
            pltpu.sync_copy(rows_v, xd_hbm.at[pl.ds(off, CH)])
            return 0
        lax.fori_loop(0, NCHUNK, body, 0)

    return k


RB = 1024
RE = 2000


def _dis_pallas(deg0, deg1):
    def body(d0, d1, dis_o):
        deg = d0[...] + d1[...] + 1.0
        dis_o[...] = jnp.where(deg > 0, lax.rsqrt(deg), 0.0)
    return pl.pallas_call(
        body,
        out_shape=jax.ShapeDtypeStruct((NP // 128, 128), jnp.float32),
    )(deg0.reshape(NP // 128, 128), deg1.reshape(NP // 128, 128))


def _mm0_pallas(x, W, dis):
    C, O = W.shape[1], W.shape[0]

    def body(x_ref, w_ref, dis_ref, o_ref):
        o_ref[...] = lax.dot_general(
            x_ref[...], w_ref[...], (((1,), (1,)), ((), ())),
            preferred_element_type=jnp.float32) * dis_ref[...]

    return pl.pallas_call(
        body,
        grid=(NP // RB,),
        in_specs=[pl.BlockSpec((RB, C), lambda i: (i, 0)),
                  pl.BlockSpec((O, C), lambda i: (0, 0)),
                  pl.BlockSpec((RB, 1), lambda i: (i, 0))],
        out_specs=pl.BlockSpec((RB, O), lambda i: (i, 0)),
        out_shape=jax.ShapeDtypeStruct((NP, O), jnp.float32),
    )(x, W, dis)


def _fused_layer_pallas(a0, a1, g, dis, b, Wnext):
    O = g.shape[1]
    On = Wnext.shape[0]

    def body(a0_ref, a1_ref, g_ref, dis_ref, b_ref, w_ref, o_ref):
        h = (a0_ref[...] + a1_ref[...] + g_ref[...]) * dis_ref[...] + b_ref[...]
        h = jnp.maximum(h, 0.0)
        o_ref[...] = lax.dot_general(
            h, w_ref[...], (((1,), (1,)), ((), ())),
            preferred_element_type=jnp.float32) * dis_ref[...]

    return pl.pallas_call(
        body,
        grid=(NP // RB,),
        in_specs=[pl.BlockSpec((RB, O), lambda i: (i, 0)),
                  pl.BlockSpec((RB, O), lambda i: (i, 0)),
                  pl.BlockSpec((RB, O), lambda i: (i, 0)),
                  pl.BlockSpec((RB, 1), lambda i: (i, 0)),
                  pl.BlockSpec((1, O), lambda i: (0, 0)),
                  pl.BlockSpec((On, O), lambda i: (0, 0))],
        out_specs=pl.BlockSpec((RB, On), lambda i: (i, 0)),
        out_shape=jax.ShapeDtypeStruct((NP, On), jnp.float32),
    )(a0, a1, g, dis, b, Wnext)


def _final_layer_pallas(a0, a1, g, dis, b, Wn0, bn0, Wn1, bn1):
    O = g.shape[1]

    def body(a0_ref, a1_ref, g_ref, dis_ref, b_ref,
             wn0_ref, bn0_ref, wn1_ref, bn1_ref, h_o, n_o):
        h = (a0_ref[...] + a1_ref[...] + g_ref[...]) * dis_ref[...] + b_ref[...]
        h = jnp.maximum(h, 0.0)
        h_o[...] = h
        n1 = lax.dot_general(h, wn0_ref[...], (((1,), (1,)), ((), ())),
                             preferred_element_type=jnp.float32)
        n1 = jnp.maximum(n1 + bn0_ref[...], 0.0)
        n2 = lax.dot_general(n1, wn1_ref[...], (((1,), (1,)), ((), ())),
                             preferred_element_type=jnp.float32)
        n_o[...] = jnp.maximum(n2 + bn1_ref[0], 0.0)

    return pl.pallas_call(
        body,
        grid=(NP // RB,),
        in_specs=[pl.BlockSpec((RB, O), lambda i: (i, 0)),
                  pl.BlockSpec((RB, O), lambda i: (i, 0)),
                  pl.BlockSpec((RB, O), lambda i: (i, 0)),
                  pl.BlockSpec((RB, 1), lambda i: (i, 0)),
                  pl.BlockSpec((1, O), lambda i: (0, 0)),
                  pl.BlockSpec((32, 32), lambda i: (0, 0)),
                  pl.BlockSpec((1, 32), lambda i: (0, 0)),
                  pl.BlockSpec((8, 32), lambda i: (0, 0)),
                  pl.BlockSpec(memory_space=pltpu.SMEM)],
        out_specs=(pl.BlockSpec((RB, O), lambda i: (i, 0)),
                   pl.BlockSpec((RB, 8), lambda i: (i, 0))),
        out_shape=(jax.ShapeDtypeStruct((NP, O), jnp.float32),
                   jax.ShapeDtypeStruct((NP, 8), jnp.float32)),
    )(a0, a1, g, dis, b, Wn0, bn0, Wn1, bn1)


def _edge_mlp_pallas(xs, xd, A1, B1, be0, We1, be1, We2, be2, We3, be3):
    def body(xs_ref, xd_ref, a1_ref, b1_ref, be0_ref, w1_ref, b1b_ref,
             w2_ref, b2_ref, w3_ref, b3_ref, o_ref):
        e = lax.dot_general(xs_ref[...], a1_ref[...], (((1,), (1,)), ((), ())),
                            preferred_element_type=jnp.float32)
        e = e + lax.dot_general(xd_ref[...], b1_ref[...],
                                (((1,), (1,)), ((), ())),
                                preferred_element_type=jnp.float32)
        e = jnp.maximum(e + be0_ref[...], 0.0)
        e = lax.dot_general(e, w1_ref[...], (((1,), (1,)), ((), ())),
                            preferred_element_type=jnp.float32)
        e = jnp.maximum(e + b1b_ref[...], 0.0)
        e = lax.dot_general(e, w2_ref[...], (((1,), (1,)), ((), ())),
                            preferred_element_type=jnp.float32)
        e = jnp.maximum(e + b2_ref[...], 0.0)
        e = lax.dot_general(e, w3_ref[...], (((1,), (1,)), ((), ())),
                            preferred_element_type=jnp.float32)
        o_ref[...] = jnp.maximum(e + b3_ref[0], 0.0)

    return pl.pallas_call(
        body,
        grid=(E // RE,),
        in_specs=[pl.BlockSpec((RE, 32), lambda i: (i, 0)),
                  pl.BlockSpec((RE, 32), lambda i: (i, 0)),
                  pl.BlockSpec((64, 32), lambda i: (0, 0)),
                  pl.BlockSpec((64, 32), lambda i: (0, 0)),
                  pl.BlockSpec((1, 64), lambda i: (0, 0)),
                  pl.BlockSpec((32, 64), lambda i: (0, 0)),
                  pl.BlockSpec((1, 32), lambda i: (0, 0)),
                  pl.BlockSpec((32, 32), lambda i: (0, 0)),
                  pl.BlockSpec((1, 32), lambda i: (0, 0)),
                  pl.BlockSpec((8, 32), lambda i: (0, 0)),
                  pl.BlockSpec(memory_space=pltpu.SMEM)],
        out_specs=pl.BlockSpec((RE, 8), lambda i: (i, 0)),
        out_shape=jax.ShapeDtypeStruct((E, 8), jnp.float32),
    )(xs, xd, A1, B1, be0, We1, be1, We2, be2, We3, be3)


def kernel(x, edge_index, edge_weight,
           Wc0, bc0, Wc1, bc1, Wc2, bc2, Wc3, bc3, Wc4, bc4,
           Wn0, bn0, Wn1, bn1,
           We0, be0, We1, be1, We2, be2, We3, be3):
    src = edge_index[0]
    dst = edge_index[1]

    deg = _deg_kernel_fn()(dst, edge_weight)
    dis = _dis_pallas(deg[0], deg[1]).reshape(NP, 1)

    xp = jnp.pad(x, ((0, NP - N), (0, 0)))

    convs = ((Wc0, bc0), (Wc1, bc1), (Wc2, bc2), (Wc3, bc3), (Wc4, bc4))
    g = _mm0_pallas(xp, Wc0, dis)
    for li in range(5):
        W, b = convs[li]
        O = W.shape[0]
        acc = _scatter_kernel_fn(O)(g, src, dst, edge_weight)
        if li < 4:
            Wnext = convs[li + 1][0]
            g = _fused_layer_pallas(acc[0], acc[1], g, dis,
                                    b.reshape(1, O), Wnext)
        else:
            h5, n_full = _final_layer_pallas(
                acc[0], acc[1], g, dis, b.reshape(1, O),
                Wn0, bn0.reshape(1, 32), jnp.pad(Wn1, ((0, 7), (0, 0))), bn1)

    xs, xd = _edge_gather_kernel_fn(32)(h5, src, dst)
    e = _edge_mlp_pallas(xs, xd,
                         We0[:, :32], We0[:, 32:], be0.reshape(1, 64),
                         We1, be1.reshape(1, 32),
                         We2, be2.reshape(1, 32),
                         jnp.pad(We3, ((0, 7), (0, 0))), be3)
    return (n_full[:N, :1], e[:, :1])
